# baseline (speedup 1.0000x reference)
import functools

import numpy as np
import jax
import jax.numpy as jnp
from jax import lax
from jax.experimental import pallas as pl
from jax.experimental.pallas import tpu as pltpu
from jax.experimental.pallas import tpu_sc as plsc

N = 10000
E = 160000
D = 128
R = 64
NB = 4
DC = NB * D

NC = 2
NS = 16
NW = NC * NS
LANES = 16

C = 64
G = 2 * ((E + 2 * NW * C - 1) // (2 * NW * C))
EP = NW * G * C

NPAD = 10240
ROWS_PT = NPAD // NS
BN = 1024
BW = 2048

_PERM = np.empty((DC,), np.int32)
for _k in range(D // LANES):
    for _h in range(2):
        for _m in range(LANES):
            for _r in range(2):
                _PERM[64 * _k + 32 * _h + 2 * _m + _r] = (
                    (2 * _h + _r) * D + LANES * _k + _m
                )


def _tc_prep_body(x_ref, bcat_ref, lw_ref, b_ref, y_ref, s_ref):
    x = x_ref[...]
    y = jnp.dot(x, bcat_ref[...], preferred_element_type=jnp.float32,
                precision=jax.lax.Precision.HIGHEST)
    y_ref[...] = y.astype(jnp.bfloat16)
    s_ref[...] = (
        jnp.dot(x, lw_ref[...], preferred_element_type=jnp.float32,
                precision=jax.lax.Precision.HIGHEST) + b_ref[...]
    )


_tc_prep = pl.pallas_call(
    _tc_prep_body,
    grid=(NPAD // BN,),
    in_specs=[
        pl.BlockSpec((BN, D), lambda i: (i, 0)),
        pl.BlockSpec((D, DC), lambda i: (0, 0)),
        pl.BlockSpec((D, D), lambda i: (0, 0)),
        pl.BlockSpec((1, D), lambda i: (0, 0)),
    ],
    out_specs=[
        pl.BlockSpec((BN, DC), lambda i: (i, 0)),
        pl.BlockSpec((BN, D), lambda i: (i, 0)),
    ],
    out_shape=[
        jax.ShapeDtypeStruct((NPAD, DC), jnp.bfloat16),
        jax.ShapeDtypeStruct((NPAD, D), jnp.float32),
    ],
)


def _tc_w_body(et_ref, nm_ref, ccat_ref, w_ref):
    et = et_ref[...]
    iot = lax.broadcasted_iota(jnp.int32, (R, BW), 0)
    oh = (iot == et).astype(jnp.float32)
    w_ref[...] = (
        jnp.dot(ccat_ref[...], oh, preferred_element_type=jnp.float32,
                precision=jax.lax.Precision.HIGHEST)
        * nm_ref[...]
    )


_tc_w = pl.pallas_call(
    _tc_w_body,
    grid=(EP // BW,),
    in_specs=[
        pl.BlockSpec((1, BW), lambda i: (0, i)),
        pl.BlockSpec((1, BW), lambda i: (0, i)),
        pl.BlockSpec((2 * NB, R), lambda i: (0, 0)),
    ],
    out_specs=pl.BlockSpec((2 * NB, BW), lambda i: (0, i)),
    out_shape=jax.ShapeDtypeStruct((2 * NB, EP), jnp.float32),
)


def _tc_mid_body(agg_ref, s_ref, bcat_ref, lw_ref, b_ref, y_ref, s2_ref):
    h = jax.nn.relu(agg_ref[0] + agg_ref[1] + s_ref[...])
    y = jnp.dot(h, bcat_ref[...], preferred_element_type=jnp.float32,
                precision=jax.lax.Precision.HIGHEST)
    y_ref[...] = y.astype(jnp.bfloat16)
    s2_ref[...] = (
        jnp.dot(h, lw_ref[...], preferred_element_type=jnp.float32,
                precision=jax.lax.Precision.HIGHEST) + b_ref[...]
    )


_tc_mid = pl.pallas_call(
    _tc_mid_body,
    grid=(NPAD // BN,),
    in_specs=[
        pl.BlockSpec((NC, BN, D), lambda i: (0, i, 0)),
        pl.BlockSpec((BN, D), lambda i: (i, 0)),
        pl.BlockSpec((D, DC), lambda i: (0, 0)),
        pl.BlockSpec((D, D), lambda i: (0, 0)),
        pl.BlockSpec((1, D), lambda i: (0, 0)),
    ],
    out_specs=[
        pl.BlockSpec((BN, DC), lambda i: (i, 0)),
        pl.BlockSpec((BN, D), lambda i: (i, 0)),
    ],
    out_shape=[
        jax.ShapeDtypeStruct((NPAD, DC), jnp.bfloat16),
        jax.ShapeDtypeStruct((NPAD, D), jnp.float32),
    ],
)


def _tc_fin_body(agg_ref, s_ref, h_ref):
    h_ref[...] = agg_ref[0] + agg_ref[1] + s_ref[...]


_tc_fin = pl.pallas_call(
    _tc_fin_body,
    grid=(NPAD // BN,),
    in_specs=[
        pl.BlockSpec((NC, BN, D), lambda i: (0, i, 0)),
        pl.BlockSpec((BN, D), lambda i: (i, 0)),
    ],
    out_specs=pl.BlockSpec((BN, D), lambda i: (i, 0)),
    out_shape=jax.ShapeDtypeStruct((NPAD, D), jnp.float32),
)


@functools.partial(
    pl.kernel,
    mesh=plsc.VectorSubcoreMesh(core_axis_name="c", subcore_axis_name="s"),
    out_type=jax.ShapeDtypeStruct((NC, NPAD, D), jnp.float32),
    scratch_types=[
        pltpu.VMEM((C,), jnp.int32),
        pltpu.VMEM((C,), jnp.int32),
        pltpu.VMEM((C,), jnp.int32),
        pltpu.VMEM((C,), jnp.int32),
        pltpu.VMEM((NB, C), jnp.float32),
        pltpu.VMEM((NB, C), jnp.float32),
        pltpu.VMEM((C, DC // 2), jnp.int32),
        pltpu.VMEM((C, DC // 2), jnp.int32),
        pltpu.VMEM((C, D), jnp.float32),
        pltpu.VMEM_SHARED((NPAD, D), jnp.float32),
        pltpu.SemaphoreType.DMA,
        pltpu.SemaphoreType.DMA,
    ],
)
def _sc_agg(ycat, srcp, dstp, wp, zeros, out,
            src0_v, src1_v, dst0_v, dst1_v, w0_v, w1_v, rows0_v, rows1_v,
            msg_v, agg_sh, sem_m, sem_r):
    cid = lax.axis_index("c")
    sid = lax.axis_index("s")
    wid = cid * NS + sid
    r0 = sid * ROWS_PT

    src_b = (src0_v, src1_v)
    dst_b = (dst0_v, dst1_v)
    w_b = (w0_v, w1_v)
    rows_b = (rows0_v, rows1_v)

    def fetch_meta(g, b):
        pltpu.async_copy(srcp.at[wid, g], src_b[b], sem_m)
        pltpu.async_copy(dstp.at[wid, g], dst_b[b], sem_m)
        pltpu.async_copy(wp.at[wid, g], w_b[b], sem_m)

    def wait_meta(g, b):
        pltpu.make_async_copy(srcp.at[wid, g], src_b[b], sem_m).wait()
        pltpu.make_async_copy(dstp.at[wid, g], dst_b[b], sem_m).wait()
        pltpu.make_async_copy(wp.at[wid, g], w_b[b], sem_m).wait()

    def start_gather(b):
        pltpu.async_copy(ycat.at[src_b[b]], rows_b[b], sem_r)

    def wait_gather(b):
        pltpu.make_async_copy(ycat.at[src_b[b]], rows_b[b], sem_r).wait()

    def compute_scatter(b):
        w_v = w_b[b]
        rows_v = rows_b[b]

        def group(t, carry2):
            base = t * LANES
            w0v = w_v[0, pl.ds(base, LANES)]
            w1v = w_v[1, pl.ds(base, LANES)]
            w2v = w_v[2, pl.ds(base, LANES)]
            w3v = w_v[3, pl.ds(base, LANES)]
            himask = jnp.int32(-65536)
            for i in range(LANES):
                j = base + i
                c0, c1, c2, c3 = w0v[i], w1v[i], w2v[i], w3v[i]
                for k in range(D // LANES):
                    v01 = rows_v[j, pl.ds(32 * k, LANES)]
                    v23 = rows_v[j, pl.ds(32 * k + LANES, LANES)]
                    y0 = lax.bitcast_convert_type(v01 << 16, jnp.float32)
                    y1 = lax.bitcast_convert_type(v01 & himask, jnp.float32)
                    y2 = lax.bitcast_convert_type(v23 << 16, jnp.float32)
                    y3 = lax.bitcast_convert_type(v23 & himask, jnp.float32)
                    msg_v[j, pl.ds(k * LANES, LANES)] = (
                        c0 * y0 + c1 * y1 + c2 * y2 + c3 * y3
                    )
            return carry2

        lax.fori_loop(0, C // LANES, group, 0)
        pltpu.sync_copy(msg_v, agg_sh.at[dst_b[b]], add=True)

    pltpu.sync_copy(zeros.at[pl.ds(r0, ROWS_PT)], agg_sh.at[pl.ds(r0, ROWS_PT)])
    plsc.subcore_barrier()

    fetch_meta(0, 0)
    wait_meta(0, 0)
    start_gather(0)
    fetch_meta(1, 1)

    def chunk_pair(i, carry):
        g0 = 2 * i
        g1 = g0 + 1
        wait_gather(0)
        wait_meta(g1, 1)
        start_gather(1)
        compute_scatter(0)

        @pl.when(g0 + 2 < G)
        def _():
            fetch_meta(g0 + 2, 0)

        wait_gather(1)

        @pl.when(g1 + 1 < G)
        def _():
            wait_meta(g1 + 1, 0)
            start_gather(0)

        compute_scatter(1)

        @pl.when(g1 + 2 < G)
        def _():
            fetch_meta(g1 + 2, 1)

        return carry

    lax.fori_loop(0, G // 2, chunk_pair, 0)
    plsc.subcore_barrier()
    pltpu.sync_copy(agg_sh.at[pl.ds(r0, ROWS_PT)],
                    out.at[cid, pl.ds(r0, ROWS_PT)])


def kernel(feats, edge_index, etype, norm, coeff1, bases1, loop_w1, bias1,
           coeff2, bases2, loop_w2, bias2):
    f32 = jnp.float32
    x = jnp.pad(feats, ((0, NPAD - N), (0, 0)))
    perm = jnp.asarray(_PERM)
    bcat1 = bases1.transpose(1, 0, 2).reshape(D, DC)[:, perm]
    bcat2 = bases2.transpose(1, 0, 2).reshape(D, DC)[:, perm]
    b1 = bias1.reshape(1, D)
    b2 = bias2.reshape(1, D)

    pad = EP - E
    pad_src = (jnp.arange(pad, dtype=jnp.int32) * 97) % N
    srcp = jnp.concatenate([edge_index[0], pad_src]).reshape(NW, G, C)
    dstp = jnp.pad(edge_index[1], (0, pad)).reshape(NW, G, C)
    etp = jnp.pad(etype, (0, pad)).reshape(1, EP)
    nmp = jnp.pad(norm[:, 0], (0, pad)).reshape(1, EP)
    ccat = jnp.concatenate([coeff1.T, coeff2.T], axis=0)
    zeros = jnp.zeros((NPAD, D), f32)

    wT = _tc_w(etp, nmp, ccat)
    w5 = wT.reshape(2, NB, NW, G, C).transpose(0, 2, 3, 1, 4)
    w1p, w2p = w5[0], w5[1]

    def as_i32(y):
        return jax.lax.bitcast_convert_type(
            y.reshape(NPAD, DC // 2, 2), jnp.int32)

    y1, s1 = _tc_prep(x, bcat1, loop_w1, b1)
    agg1 = _sc_agg(as_i32(y1), srcp, dstp, w1p, zeros)
    y2, s2 = _tc_mid(agg1, s1, bcat2, loop_w2, b2)
    agg2 = _sc_agg(as_i32(y2), srcp, dstp, w2p, zeros)
    h2 = _tc_fin(agg2, s2)
    return h2[:N]


# speedup vs baseline: 1.1658x; 1.1658x over previous
import functools

import jax
import jax.numpy as jnp
from jax import lax
from jax.experimental import pallas as pl
from jax.experimental.pallas import tpu as pltpu
from jax.experimental.pallas import tpu_sc as plsc

N = 10000
E = 160000
D = 128
R = 64
NB = 4
DC = NB * D

NC = 2
NS = 16
NW = NC * NS
LANES = 16

C = 64
G = (E + NW * C - 1) // (NW * C)
EP = NW * G * C

NPAD = 10240
ROWS_PT = NPAD // NS
BN = 1024
BW = 2048


def _tc_prep_body(x_ref, bcat_ref, lw_ref, b_ref, y_ref, s_ref):
    x = x_ref[...]
    y_ref[...] = jnp.dot(x, bcat_ref[...], preferred_element_type=jnp.float32,
                         precision=jax.lax.Precision.HIGHEST)
    s_ref[...] = (
        jnp.dot(x, lw_ref[...], preferred_element_type=jnp.float32,
                precision=jax.lax.Precision.HIGHEST) + b_ref[...]
    )


_tc_prep = pl.pallas_call(
    _tc_prep_body,
    grid=(NPAD // BN,),
    in_specs=[
        pl.BlockSpec((BN, D), lambda i: (i, 0)),
        pl.BlockSpec((D, DC), lambda i: (0, 0)),
        pl.BlockSpec((D, D), lambda i: (0, 0)),
        pl.BlockSpec((1, D), lambda i: (0, 0)),
    ],
    out_specs=[
        pl.BlockSpec((BN, DC), lambda i: (i, 0)),
        pl.BlockSpec((BN, D), lambda i: (i, 0)),
    ],
    out_shape=[
        jax.ShapeDtypeStruct((NPAD, DC), jnp.float32),
        jax.ShapeDtypeStruct((NPAD, D), jnp.float32),
    ],
)


def _tc_w_body(et_ref, nm_ref, ccat_ref, w_ref):
    et = et_ref[...]
    iot = lax.broadcasted_iota(jnp.int32, (R, BW), 0)
    oh = (iot == et).astype(jnp.float32)
    w_ref[...] = (
        jnp.dot(ccat_ref[...], oh, preferred_element_type=jnp.float32,
                precision=jax.lax.Precision.HIGHEST)
        * nm_ref[...]
    )


_tc_w = pl.pallas_call(
    _tc_w_body,
    grid=(EP // BW,),
    in_specs=[
        pl.BlockSpec((1, BW), lambda i: (0, i)),
        pl.BlockSpec((1, BW), lambda i: (0, i)),
        pl.BlockSpec((2 * NB, R), lambda i: (0, 0)),
    ],
    out_specs=pl.BlockSpec((2 * NB, BW), lambda i: (0, i)),
    out_shape=jax.ShapeDtypeStruct((2 * NB, EP), jnp.float32),
)


def _tc_mid_body(agg_ref, s_ref, bcat_ref, lw_ref, b_ref, y_ref, s2_ref):
    h = jax.nn.relu(agg_ref[0] + agg_ref[1] + s_ref[...])
    y_ref[...] = jnp.dot(h, bcat_ref[...], preferred_element_type=jnp.float32,
                         precision=jax.lax.Precision.HIGHEST)
    s2_ref[...] = (
        jnp.dot(h, lw_ref[...], preferred_element_type=jnp.float32,
                precision=jax.lax.Precision.HIGHEST) + b_ref[...]
    )


_tc_mid = pl.pallas_call(
    _tc_mid_body,
    grid=(NPAD // BN,),
    in_specs=[
        pl.BlockSpec((NC, BN, D), lambda i: (0, i, 0)),
        pl.BlockSpec((BN, D), lambda i: (i, 0)),
        pl.BlockSpec((D, DC), lambda i: (0, 0)),
        pl.BlockSpec((D, D), lambda i: (0, 0)),
        pl.BlockSpec((1, D), lambda i: (0, 0)),
    ],
    out_specs=[
        pl.BlockSpec((BN, DC), lambda i: (i, 0)),
        pl.BlockSpec((BN, D), lambda i: (i, 0)),
    ],
    out_shape=[
        jax.ShapeDtypeStruct((NPAD, DC), jnp.float32),
        jax.ShapeDtypeStruct((NPAD, D), jnp.float32),
    ],
)


def _tc_fin_body(agg_ref, s_ref, h_ref):
    h_ref[...] = agg_ref[0] + agg_ref[1] + s_ref[...]


_tc_fin = pl.pallas_call(
    _tc_fin_body,
    grid=(NPAD // BN,),
    in_specs=[
        pl.BlockSpec((NC, BN, D), lambda i: (0, i, 0)),
        pl.BlockSpec((BN, D), lambda i: (i, 0)),
    ],
    out_specs=pl.BlockSpec((BN, D), lambda i: (i, 0)),
    out_shape=jax.ShapeDtypeStruct((NPAD, D), jnp.float32),
)


@functools.partial(
    pl.kernel,
    mesh=plsc.VectorSubcoreMesh(core_axis_name="c", subcore_axis_name="s"),
    out_type=jax.ShapeDtypeStruct((NC, NPAD, D), jnp.float32),
    scratch_types=[
        pltpu.VMEM((C,), jnp.int32),
        pltpu.VMEM((C,), jnp.int32),
        pltpu.VMEM((NB, C), jnp.float32),
        pltpu.VMEM((C, DC), jnp.float32),
        pltpu.VMEM((C, D), jnp.float32),
        pltpu.VMEM_SHARED((NPAD, D), jnp.float32),
        pltpu.SemaphoreType.DMA,
    ],
)
def _sc_agg(ycat, srcp, dstp, wp, zeros, out,
            src_v, dst_v, w_v, rows_v, msg_v, agg_sh, sem):
    cid = lax.axis_index("c")
    sid = lax.axis_index("s")
    wid = cid * NS + sid
    r0 = sid * ROWS_PT

    pltpu.sync_copy(zeros.at[pl.ds(r0, ROWS_PT)], agg_sh.at[pl.ds(r0, ROWS_PT)])
    plsc.subcore_barrier()

    def chunk(g, carry):
        pltpu.sync_copy(srcp.at[wid, g], src_v)
        pltpu.sync_copy(dstp.at[wid, g], dst_v)
        pltpu.sync_copy(wp.at[wid, g], w_v)
        pltpu.async_copy(ycat.at[src_v], rows_v, sem).wait()

        def group(t, carry2):
            base = t * LANES
            w0v = w_v[0, pl.ds(base, LANES)]
            w1v = w_v[1, pl.ds(base, LANES)]
            w2v = w_v[2, pl.ds(base, LANES)]
            w3v = w_v[3, pl.ds(base, LANES)]
            for i in range(LANES):
                j = base + i
                c0, c1, c2, c3 = w0v[i], w1v[i], w2v[i], w3v[i]
                for k in range(D // LANES):
                    o = k * LANES
                    msg_v[j, pl.ds(o, LANES)] = (
                        c0 * rows_v[j, pl.ds(o, LANES)]
                        + c1 * rows_v[j, pl.ds(D + o, LANES)]
                        + c2 * rows_v[j, pl.ds(2 * D + o, LANES)]
                        + c3 * rows_v[j, pl.ds(3 * D + o, LANES)]
                    )
            return carry2

        lax.fori_loop(0, C // LANES, group, 0)
        pltpu.sync_copy(msg_v, agg_sh.at[dst_v], add=True)
        return carry

    lax.fori_loop(0, G, chunk, 0)
    plsc.subcore_barrier()
    pltpu.sync_copy(agg_sh.at[pl.ds(r0, ROWS_PT)],
                    out.at[cid, pl.ds(r0, ROWS_PT)])


def kernel(feats, edge_index, etype, norm, coeff1, bases1, loop_w1, bias1,
           coeff2, bases2, loop_w2, bias2):
    f32 = jnp.float32
    x = jnp.pad(feats, ((0, NPAD - N), (0, 0)))
    bcat1 = bases1.transpose(1, 0, 2).reshape(D, DC)
    bcat2 = bases2.transpose(1, 0, 2).reshape(D, DC)
    b1 = bias1.reshape(1, D)
    b2 = bias2.reshape(1, D)

    pad = EP - E
    pad_src = (jnp.arange(pad, dtype=jnp.int32) * 97) % N
    srcp = jnp.concatenate([edge_index[0], pad_src]).reshape(NW, G, C)
    dstp = jnp.pad(edge_index[1], (0, pad)).reshape(NW, G, C)
    etp = jnp.pad(etype, (0, pad)).reshape(1, EP)
    nmp = jnp.pad(norm[:, 0], (0, pad)).reshape(1, EP)
    ccat = jnp.concatenate([coeff1.T, coeff2.T], axis=0)
    zeros = jnp.zeros((NPAD, D), f32)

    wT = _tc_w(etp, nmp, ccat)
    w5 = wT.reshape(2, NB, NW, G, C).transpose(0, 2, 3, 1, 4)
    w1p, w2p = w5[0], w5[1]

    y1, s1 = _tc_prep(x, bcat1, loop_w1, b1)
    agg1 = _sc_agg(y1, srcp, dstp, w1p, zeros)
    y2, s2 = _tc_mid(agg1, s1, bcat2, loop_w2, b2)
    agg2 = _sc_agg(y2, srcp, dstp, w2p, zeros)
    h2 = _tc_fin(agg2, s2)
    return h2[:N]
